# Initial kernel scaffold; baseline (speedup 1.0000x reference)
import jax
import jax.numpy as jnp
from jax.experimental import pallas as pl


def kernel(x, perm):
    raise NotImplementedError("write your pallas kernel here")


# speedup vs baseline: 3.6986x; 3.6986x over previous
import jax
import jax.numpy as jnp
from jax.experimental import pallas as pl
from jax.experimental.pallas import tpu as pltpu

N = 262144
DIM = 128
BLOCK = 2048


def _body(perm_ref, x_ref, out_ref):
    idx = jnp.broadcast_to(perm_ref[...], (BLOCK, DIM))
    out_ref[...] = jnp.take_along_axis(x_ref[...], idx, axis=1)


def kernel(x, perm):
    grid = N // BLOCK
    return pl.pallas_call(
        _body,
        grid=(grid,),
        in_specs=[
            pl.BlockSpec((1, DIM), lambda i: (0, 0)),
            pl.BlockSpec((BLOCK, DIM), lambda i: (i, 0)),
        ],
        out_specs=pl.BlockSpec((BLOCK, DIM), lambda i: (i, 0)),
        out_shape=jax.ShapeDtypeStruct((N, DIM), jnp.float32),
        compiler_params=pltpu.CompilerParams(
            dimension_semantics=("arbitrary",),
        ),
    )(perm.reshape(1, DIM), x)
